# Initial kernel scaffold; baseline (speedup 1.0000x reference)
import functools

import jax
import jax.numpy as jnp
from jax import lax
from jax.experimental import pallas as pl
from jax.experimental.pallas import tpu as pltpu
from jax.experimental.pallas import tpu_sc as plsc

N = 10000
F = 128
E_RAW = 320000
E_TOT = E_RAW + N
NW = 32
CH = 128
NCH = 81
EPT = NCH * CH
E_PAD = EPT * NW
ACC_N = 10240
RPT = ACC_N // 16


def _dense1_body(x_ref, wg_ref, as_ref, ad_ref, wr1_ref, h_ref, av_ref, res_ref):
    x = x_ref[...]
    h = lax.dot_general(x, wg_ref[...], (((1,), (1,)), ((), ())),
                        preferred_element_type=jnp.float32)
    h_ref[...] = h
    av_ref[0, :] = jnp.sum(h * as_ref[...][None, :], axis=1)
    av_ref[1, :] = jnp.sum(h * ad_ref[...][None, :], axis=1)
    res_ref[...] = lax.dot_general(x, wr1_ref[...], (((1,), (1,)), ((), ())),
                                   preferred_element_type=jnp.float32)


_dense1 = pl.pallas_call(
    _dense1_body,
    out_shape=[
        jax.ShapeDtypeStruct((N, F), jnp.float32),
        jax.ShapeDtypeStruct((2, N), jnp.float32),
        jax.ShapeDtypeStruct((N, F), jnp.float32),
    ],
)


def _mid_body(acc_ref, den_ref, bg_ref, res1_ref, g1_ref, be1_ref, wtr_ref,
              astr_ref, adtr_ref, wr2_ref, h1_ref, h2p_ref, av2_ref, res2_ref):
    agg = acc_ref[0, :N, :] + acc_ref[1, :N, :]
    den = den_ref[0, :N] + den_ref[1, :N]
    t = agg / den[:, None] + bg_ref[...][None, :] + res1_ref[...]
    mu = jnp.mean(t, axis=0)
    var = jnp.mean((t - mu[None, :]) ** 2, axis=0)
    bn = (g1_ref[...][None, :] * (t - mu[None, :])
          / jnp.sqrt(var + 1e-5)[None, :] + be1_ref[...][None, :])
    h1 = bn + res1_ref[...]
    h1_ref[...] = h1
    h2p = lax.dot_general(h1, wtr_ref[...], (((1,), (1,)), ((), ())),
                          preferred_element_type=jnp.float32)
    h2p_ref[...] = h2p
    av2_ref[0, :] = jnp.sum(h2p * astr_ref[...][None, :], axis=1)
    av2_ref[1, :] = jnp.sum(h2p * adtr_ref[...][None, :], axis=1)
    res2_ref[...] = lax.dot_general(h1, wr2_ref[...], (((1,), (1,)), ((), ())),
                                    preferred_element_type=jnp.float32)


_mid = pl.pallas_call(
    _mid_body,
    out_shape=[
        jax.ShapeDtypeStruct((N, F), jnp.float32),
        jax.ShapeDtypeStruct((N, F), jnp.float32),
        jax.ShapeDtypeStruct((2, N), jnp.float32),
        jax.ShapeDtypeStruct((N, F), jnp.float32),
    ],
)


def _final_body(acc_ref, den_ref, btr_ref, res2_ref, g2_ref, be2_ref, h1_ref,
                wp_ref, bp_ref, out_ref):
    agg = acc_ref[0, :N, :] + acc_ref[1, :N, :]
    den = den_ref[0, :N] + den_ref[1, :N]
    t = agg / den[:, None] + btr_ref[...][None, :] + res2_ref[...]
    mu = jnp.mean(t, axis=0)
    var = jnp.mean((t - mu[None, :]) ** 2, axis=0)
    h2 = (g2_ref[...][None, :] * (t - mu[None, :])
          / jnp.sqrt(var + 1e-5)[None, :] + be2_ref[...][None, :]
          + res2_ref[...])
    hcat = jnp.concatenate([h1_ref[...], h2], axis=1)
    z = lax.dot_general(hcat, wp_ref[...], (((1,), (1,)), ((), ())),
                        preferred_element_type=jnp.float32) + bp_ref[...][None, :]
    out_ref[...] = jnp.where(z > 0, z, jnp.expm1(z))


_final = pl.pallas_call(
    _final_body,
    out_shape=jax.ShapeDtypeStruct((N, F), jnp.float32),
)


@functools.partial(
    pl.kernel,
    mesh=plsc.VectorSubcoreMesh(core_axis_name="c", subcore_axis_name="s"),
    out_type=[
        jax.ShapeDtypeStruct((2, ACC_N, F), jnp.float32),
        jax.ShapeDtypeStruct((2, ACC_N), jnp.float32),
    ],
    scratch_types=[
        pltpu.VMEM((NCH, CH), jnp.int32),
        pltpu.VMEM((NCH, CH), jnp.int32),
        pltpu.VMEM((2, N), jnp.float32),
        pltpu.VMEM((NCH, CH), jnp.float32),
        pltpu.VMEM((CH, F), jnp.float32),
        pltpu.VMEM((RPT,), jnp.float32),
        pltpu.VMEM_SHARED((ACC_N, F), jnp.float32),
        pltpu.VMEM_SHARED((ACC_N,), jnp.float32),
        pltpu.SemaphoreType.DMA,
    ],
)
def _sc_aggregate(src_hbm, dst_hbm, av_hbm, h_hbm, acc_out, den_out,
                  src_v, dst_v, av_v, e_v, rows_v, zbuf_v, acc_sh, den_sh, sem):
    c = lax.axis_index("c")
    s = lax.axis_index("s")
    wid = c * 16 + s
    ebase = wid * EPT
    rbase = s * RPT

    zero16 = jnp.zeros((16,), jnp.float32)

    def zrow(i, carry):
        for k in range(8):
            rows_v[i, pl.ds(k * 16, 16)] = zero16
        return carry

    lax.fori_loop(0, CH, zrow, 0)

    def zb(i, carry):
        zbuf_v[pl.ds(i * 16, 16)] = zero16
        return carry

    lax.fori_loop(0, RPT // 16, zb, 0)

    def zacc(k, carry):
        pltpu.sync_copy(rows_v, acc_sh.at[pl.ds(rbase + k * CH, CH)])
        return carry

    lax.fori_loop(0, RPT // CH, zacc, 0)
    pltpu.sync_copy(zbuf_v, den_sh.at[pl.ds(rbase, RPT)])

    pltpu.sync_copy(src_hbm.at[wid], src_v)
    pltpu.sync_copy(dst_hbm.at[wid], dst_v)
    pltpu.sync_copy(av_hbm, av_v)

    plsc.subcore_barrier()

    lanes = lax.broadcasted_iota(jnp.int32, (16,), 0)
    zeros16i = jnp.zeros((16,), jnp.int32)
    ones16i = jnp.ones((16,), jnp.int32)

    def pass_a(j, carry):
        for i in range(8):
            sl = pl.ds(i * 16, 16)
            sidx = src_v[j, sl]
            didx = dst_v[j, sl]
            a = (plsc.load_gather(av_v, [zeros16i, sidx])
                 + plsc.load_gather(av_v, [ones16i, didx]))
            a = jnp.maximum(a, 0.2 * a)
            e = jnp.exp(a)
            gid = ebase + j * CH + i * 16 + lanes
            e_v[j, sl] = jnp.where(gid < E_TOT, e, 0.0)
        return carry

    lax.fori_loop(0, NCH, pass_a, 0)

    def pass_b(j, carry):
        pltpu.async_copy(h_hbm.at[src_v.at[j]], rows_v, sem).wait()

        def scale(i, icarry):
            es = plsc.load_gather(e_v, [jnp.full((16,), j, jnp.int32),
                                        jnp.full((16,), i, jnp.int32)])
            for k in range(8):
                sl = pl.ds(k * 16, 16)
                rows_v[i, sl] = rows_v[i, sl] * es
            return icarry

        lax.fori_loop(0, CH, scale, 0)
        pltpu.sync_copy(rows_v, acc_sh.at[dst_v.at[j]], add=True)
        pltpu.sync_copy(e_v.at[j], den_sh.at[dst_v.at[j]], add=True)
        return carry

    lax.fori_loop(0, NCH, pass_b, 0)

    plsc.subcore_barrier()

    def cpout(k, carry):
        pltpu.sync_copy(acc_sh.at[pl.ds(rbase + k * CH, CH)], rows_v)
        pltpu.sync_copy(rows_v, acc_out.at[c, pl.ds(rbase + k * CH, CH)])
        return carry

    lax.fori_loop(0, RPT // CH, cpout, 0)
    pltpu.sync_copy(den_sh.at[pl.ds(rbase, RPT)], zbuf_v)
    pltpu.sync_copy(zbuf_v, den_out.at[c, pl.ds(rbase, RPT)])


def kernel(x, edge_index, W_gcn, a_s_gcn, a_d_gcn, b_gcn, W_r1, g1, be1,
           W_tr, a_s_tr, a_d_tr, b_tr, W_r2, g2, be2, W_p, b_p):
    loop = jnp.arange(N, dtype=jnp.int32)
    pad = (E_TOT + jnp.arange(E_PAD - E_TOT, dtype=jnp.int32)) % N
    src = jnp.concatenate([edge_index[0].astype(jnp.int32), loop, pad])
    dst = jnp.concatenate([edge_index[1].astype(jnp.int32), loop, pad])
    src3 = src.reshape(NW, NCH, CH)
    dst3 = dst.reshape(NW, NCH, CH)

    h1p, av1, res1 = _dense1(x, W_gcn, a_s_gcn, a_d_gcn, W_r1)
    acc1, den1 = _sc_aggregate(src3, dst3, av1, h1p)
    h1, h2p, av2, res2 = _mid(acc1, den1, b_gcn, res1, g1, be1,
                              W_tr, a_s_tr, a_d_tr, W_r2)
    acc2, den2 = _sc_aggregate(src3, dst3, av2, h2p)
    return _final(acc2, den2, b_tr, res2, g2, be2, h1, W_p, b_p)


# speedup vs baseline: 28.2098x; 28.2098x over previous
import functools

import jax
import jax.numpy as jnp
from jax import lax
from jax.experimental import pallas as pl
from jax.experimental.pallas import tpu as pltpu
from jax.experimental.pallas import tpu_sc as plsc

N = 10000
F = 128
E_RAW = 320000
E_TOT = E_RAW + N
NW = 32
CH = 128
NCH = 81
EPT = NCH * CH
E_PAD = EPT * NW
ACC_N = 10240
RPT = ACC_N // 16


def _dense1_body(x_ref, wg_ref, as_ref, ad_ref, wr1_ref, h_ref, av_ref, res_ref):
    x = x_ref[...]
    h = lax.dot_general(x, wg_ref[...], (((1,), (1,)), ((), ())),
                        preferred_element_type=jnp.float32)
    h_ref[...] = h
    av_ref[0, :] = jnp.sum(h * as_ref[...][None, :], axis=1)
    av_ref[1, :] = jnp.sum(h * ad_ref[...][None, :], axis=1)
    res_ref[...] = lax.dot_general(x, wr1_ref[...], (((1,), (1,)), ((), ())),
                                   preferred_element_type=jnp.float32)


_dense1 = pl.pallas_call(
    _dense1_body,
    out_shape=[
        jax.ShapeDtypeStruct((N, F), jnp.float32),
        jax.ShapeDtypeStruct((2, N), jnp.float32),
        jax.ShapeDtypeStruct((N, F), jnp.float32),
    ],
)


def _mid_body(acc_ref, den_ref, bg_ref, res1_ref, g1_ref, be1_ref, wtr_ref,
              astr_ref, adtr_ref, wr2_ref, h1_ref, h2p_ref, av2_ref, res2_ref):
    agg = acc_ref[0, :N, :] + acc_ref[1, :N, :]
    den = den_ref[0, :N] + den_ref[1, :N]
    t = agg / den[:, None] + bg_ref[...][None, :] + res1_ref[...]
    mu = jnp.mean(t, axis=0)
    var = jnp.mean((t - mu[None, :]) ** 2, axis=0)
    bn = (g1_ref[...][None, :] * (t - mu[None, :])
          / jnp.sqrt(var + 1e-5)[None, :] + be1_ref[...][None, :])
    h1 = bn + res1_ref[...]
    h1_ref[...] = h1
    h2p = lax.dot_general(h1, wtr_ref[...], (((1,), (1,)), ((), ())),
                          preferred_element_type=jnp.float32)
    h2p_ref[...] = h2p
    av2_ref[0, :] = jnp.sum(h2p * astr_ref[...][None, :], axis=1)
    av2_ref[1, :] = jnp.sum(h2p * adtr_ref[...][None, :], axis=1)
    res2_ref[...] = lax.dot_general(h1, wr2_ref[...], (((1,), (1,)), ((), ())),
                                    preferred_element_type=jnp.float32)


_mid = pl.pallas_call(
    _mid_body,
    out_shape=[
        jax.ShapeDtypeStruct((N, F), jnp.float32),
        jax.ShapeDtypeStruct((N, F), jnp.float32),
        jax.ShapeDtypeStruct((2, N), jnp.float32),
        jax.ShapeDtypeStruct((N, F), jnp.float32),
    ],
)


def _final_body(acc_ref, den_ref, btr_ref, res2_ref, g2_ref, be2_ref, h1_ref,
                wp_ref, bp_ref, out_ref):
    agg = acc_ref[0, :N, :] + acc_ref[1, :N, :]
    den = den_ref[0, :N] + den_ref[1, :N]
    t = agg / den[:, None] + btr_ref[...][None, :] + res2_ref[...]
    mu = jnp.mean(t, axis=0)
    var = jnp.mean((t - mu[None, :]) ** 2, axis=0)
    h2 = (g2_ref[...][None, :] * (t - mu[None, :])
          / jnp.sqrt(var + 1e-5)[None, :] + be2_ref[...][None, :]
          + res2_ref[...])
    hcat = jnp.concatenate([h1_ref[...], h2], axis=1)
    z = lax.dot_general(hcat, wp_ref[...], (((1,), (1,)), ((), ())),
                        preferred_element_type=jnp.float32) + bp_ref[...][None, :]
    out_ref[...] = jnp.where(z > 0, z, jnp.exp(z) - 1.0)


_final = pl.pallas_call(
    _final_body,
    out_shape=jax.ShapeDtypeStruct((N, F), jnp.float32),
)


@functools.partial(
    pl.kernel,
    mesh=plsc.VectorSubcoreMesh(core_axis_name="c", subcore_axis_name="s"),
    compiler_params=pltpu.CompilerParams(needs_layout_passes=False),
    out_type=[
        jax.ShapeDtypeStruct((2, ACC_N, F), jnp.float32),
        jax.ShapeDtypeStruct((2, ACC_N), jnp.float32),
    ],
    scratch_types=[
        pltpu.VMEM((CH,), jnp.int32),
        pltpu.VMEM((CH,), jnp.int32),
        pltpu.VMEM((N,), jnp.float32),
        pltpu.VMEM((N,), jnp.float32),
        pltpu.VMEM((CH,), jnp.float32),
        pltpu.VMEM((CH, F), jnp.float32),
        pltpu.VMEM((RPT,), jnp.float32),
        pltpu.VMEM_SHARED((ACC_N, F), jnp.float32),
        pltpu.VMEM_SHARED((ACC_N,), jnp.float32),
        pltpu.SemaphoreType.DMA,
    ],
)
def _sc_aggregate(src_hbm, dst_hbm, av_hbm, h_hbm, acc_out, den_out,
                  sidx_v, didx_v, as_v, ad_v, e_v, rows_v, zbuf_v,
                  acc_sh, den_sh, sem):
    c = lax.axis_index("c")
    s = lax.axis_index("s")
    wid = c * 16 + s
    ebase = wid * EPT
    rbase = s * RPT

    zero16 = jnp.zeros((16,), jnp.float32)

    def zrow(i, carry):
        for k in range(8):
            rows_v[i, pl.ds(k * 16, 16)] = zero16
        return carry

    lax.fori_loop(0, CH, zrow, 0)

    def zb(i, carry):
        zbuf_v[pl.ds(i * 16, 16)] = zero16
        return carry

    lax.fori_loop(0, RPT // 16, zb, 0)

    def zacc(k, carry):
        pltpu.sync_copy(rows_v, acc_sh.at[pl.ds(rbase + k * CH, CH)])
        return carry

    lax.fori_loop(0, RPT // CH, zacc, 0)
    pltpu.sync_copy(zbuf_v, den_sh.at[pl.ds(rbase, RPT)])

    pltpu.sync_copy(av_hbm.at[0], as_v)
    pltpu.sync_copy(av_hbm.at[1], ad_v)

    plsc.subcore_barrier()

    lanes = lax.broadcasted_iota(jnp.int32, (16,), 0)

    def chunk(j, carry):
        pltpu.sync_copy(src_hbm.at[wid, j], sidx_v)
        pltpu.sync_copy(dst_hbm.at[wid, j], didx_v)
        gcp = pltpu.async_copy(h_hbm.at[sidx_v], rows_v, sem)

        for i in range(8):
            sl = pl.ds(i * 16, 16)
            a = (plsc.load_gather(as_v, [sidx_v[sl]])
                 + plsc.load_gather(ad_v, [didx_v[sl]]))
            a = jnp.maximum(a, 0.2 * a)
            e = jnp.exp(a)
            gid = ebase + j * CH + i * 16 + lanes
            e_v[sl] = jnp.where(gid < E_TOT, e, 0.0)

        gcp.wait()

        def scale(i, icarry):
            es = plsc.load_gather(e_v, [jnp.full((16,), i, jnp.int32)])
            for k in range(8):
                sl = pl.ds(k * 16, 16)
                rows_v[i, sl] = rows_v[i, sl] * es
            return icarry

        lax.fori_loop(0, CH, scale, 0)
        pltpu.sync_copy(rows_v, acc_sh.at[didx_v], add=True)
        pltpu.sync_copy(e_v, den_sh.at[didx_v], add=True)
        return carry

    lax.fori_loop(0, NCH, chunk, 0)

    plsc.subcore_barrier()

    def cpout(k, carry):
        pltpu.sync_copy(acc_sh.at[pl.ds(rbase + k * CH, CH)], rows_v)
        pltpu.sync_copy(rows_v, acc_out.at[c, pl.ds(rbase + k * CH, CH)])
        return carry

    lax.fori_loop(0, RPT // CH, cpout, 0)
    pltpu.sync_copy(den_sh.at[pl.ds(rbase, RPT)], zbuf_v)
    pltpu.sync_copy(zbuf_v, den_out.at[c, pl.ds(rbase, RPT)])


def kernel(x, edge_index, W_gcn, a_s_gcn, a_d_gcn, b_gcn, W_r1, g1, be1,
           W_tr, a_s_tr, a_d_tr, b_tr, W_r2, g2, be2, W_p, b_p):
    loop = jnp.arange(N, dtype=jnp.int32)
    pad = (E_TOT + jnp.arange(E_PAD - E_TOT, dtype=jnp.int32)) % N
    src = jnp.concatenate([edge_index[0].astype(jnp.int32), loop, pad])
    dst = jnp.concatenate([edge_index[1].astype(jnp.int32), loop, pad])
    src3 = src.reshape(NW, NCH, CH)
    dst3 = dst.reshape(NW, NCH, CH)

    h1p, av1, res1 = _dense1(x, W_gcn, a_s_gcn, a_d_gcn, W_r1)
    acc1, den1 = _sc_aggregate(src3, dst3, av1, h1p)
    h1, h2p, av2, res2 = _mid(acc1, den1, b_gcn, res1, g1, be1,
                              W_tr, a_s_tr, a_d_tr, W_r2)
    acc2, den2 = _sc_aggregate(src3, dst3, av2, h2p)
    return _final(acc2, den2, b_tr, res2, g2, be2, h1, W_p, b_p)


# speedup vs baseline: 52.4765x; 1.8602x over previous
import functools

import jax
import jax.numpy as jnp
from jax import lax
from jax.experimental import pallas as pl
from jax.experimental.pallas import tpu as pltpu
from jax.experimental.pallas import tpu_sc as plsc

N = 10000
F = 128
E_RAW = 320000
E_TOT = E_RAW + N
NW = 32
CH = 96
NCH = 108
CPR = 64
EPT = NCH * CH
E_PAD = EPT * NW
ACC_N = 10240
RPT = ACC_N // 16


def _dense1_body(x_ref, wg_ref, as_ref, ad_ref, wr1_ref, h_ref, av_ref, res_ref):
    x = x_ref[...]
    h = lax.dot_general(x, wg_ref[...], (((1,), (1,)), ((), ())),
                        preferred_element_type=jnp.float32)
    h_ref[...] = h
    av_ref[0, :] = jnp.sum(h * as_ref[...][None, :], axis=1)
    av_ref[1, :] = jnp.sum(h * ad_ref[...][None, :], axis=1)
    res_ref[...] = lax.dot_general(x, wr1_ref[...], (((1,), (1,)), ((), ())),
                                   preferred_element_type=jnp.float32)


_dense1 = pl.pallas_call(
    _dense1_body,
    out_shape=[
        jax.ShapeDtypeStruct((N, F), jnp.float32),
        jax.ShapeDtypeStruct((2, N), jnp.float32),
        jax.ShapeDtypeStruct((N, F), jnp.float32),
    ],
)


def _mid_body(acc_ref, den_ref, bg_ref, res1_ref, g1_ref, be1_ref, wtr_ref,
              astr_ref, adtr_ref, wr2_ref, h1_ref, h2p_ref, av2_ref, res2_ref):
    agg = acc_ref[0, :N, :] + acc_ref[1, :N, :]
    den = den_ref[0, :N] + den_ref[1, :N]
    t = agg / den[:, None] + bg_ref[...][None, :] + res1_ref[...]
    mu = jnp.mean(t, axis=0)
    var = jnp.mean((t - mu[None, :]) ** 2, axis=0)
    bn = (g1_ref[...][None, :] * (t - mu[None, :])
          / jnp.sqrt(var + 1e-5)[None, :] + be1_ref[...][None, :])
    h1 = bn + res1_ref[...]
    h1_ref[...] = h1
    h2p = lax.dot_general(h1, wtr_ref[...], (((1,), (1,)), ((), ())),
                          preferred_element_type=jnp.float32)
    h2p_ref[...] = h2p
    av2_ref[0, :] = jnp.sum(h2p * astr_ref[...][None, :], axis=1)
    av2_ref[1, :] = jnp.sum(h2p * adtr_ref[...][None, :], axis=1)
    res2_ref[...] = lax.dot_general(h1, wr2_ref[...], (((1,), (1,)), ((), ())),
                                    preferred_element_type=jnp.float32)


_mid = pl.pallas_call(
    _mid_body,
    out_shape=[
        jax.ShapeDtypeStruct((N, F), jnp.float32),
        jax.ShapeDtypeStruct((N, F), jnp.float32),
        jax.ShapeDtypeStruct((2, N), jnp.float32),
        jax.ShapeDtypeStruct((N, F), jnp.float32),
    ],
)


def _final_body(acc_ref, den_ref, btr_ref, res2_ref, g2_ref, be2_ref, h1_ref,
                wp_ref, bp_ref, out_ref):
    agg = acc_ref[0, :N, :] + acc_ref[1, :N, :]
    den = den_ref[0, :N] + den_ref[1, :N]
    t = agg / den[:, None] + btr_ref[...][None, :] + res2_ref[...]
    mu = jnp.mean(t, axis=0)
    var = jnp.mean((t - mu[None, :]) ** 2, axis=0)
    h2 = (g2_ref[...][None, :] * (t - mu[None, :])
          / jnp.sqrt(var + 1e-5)[None, :] + be2_ref[...][None, :]
          + res2_ref[...])
    hcat = jnp.concatenate([h1_ref[...], h2], axis=1)
    z = lax.dot_general(hcat, wp_ref[...], (((1,), (1,)), ((), ())),
                        preferred_element_type=jnp.float32) + bp_ref[...][None, :]
    out_ref[...] = jnp.where(z > 0, z, jnp.exp(z) - 1.0)


_final = pl.pallas_call(
    _final_body,
    out_shape=jax.ShapeDtypeStruct((N, F), jnp.float32),
)


@functools.partial(
    pl.kernel,
    mesh=plsc.VectorSubcoreMesh(core_axis_name="c", subcore_axis_name="s"),
    compiler_params=pltpu.CompilerParams(needs_layout_passes=False),
    out_type=[
        jax.ShapeDtypeStruct((2, ACC_N, F), jnp.float32),
        jax.ShapeDtypeStruct((2, ACC_N), jnp.float32),
    ],
    scratch_types=[
        pltpu.VMEM((N,), jnp.float32),
        pltpu.VMEM((N,), jnp.float32),
        pltpu.VMEM((CH,), jnp.int32),
        pltpu.VMEM((CH,), jnp.int32),
        pltpu.VMEM((CH,), jnp.int32),
        pltpu.VMEM((CH,), jnp.int32),
        pltpu.VMEM((2 * CH,), jnp.float32),
        pltpu.VMEM((2, CH, F), jnp.float32),
        pltpu.VMEM((RPT,), jnp.float32),
        pltpu.VMEM_SHARED((ACC_N, F), jnp.float32),
        pltpu.VMEM_SHARED((ACC_N,), jnp.float32),
        pltpu.SemaphoreType.DMA((2,)),
        pltpu.SemaphoreType.DMA((2,)),
        pltpu.SemaphoreType.DMA((2,)),
        pltpu.SemaphoreType.DMA((2,)),
        pltpu.SemaphoreType.DMA((2,)),
    ],
)
def _sc_aggregate(src_hbm, dst_hbm, av_hbm, h_hbm, acc_out, den_out,
                  as_v, ad_v, sidx0, sidx1, didx0, didx1, e_v, rows_v, zbuf_v,
                  acc_sh, den_sh, sem_is, sem_id, sem_g, sem_s, sem_d):
    c = lax.axis_index("c")
    s = lax.axis_index("s")
    wid = c * 16 + s
    ebase = wid * EPT
    rbase = s * RPT
    sidx = (sidx0, sidx1)
    didx = (didx0, didx1)

    zero16 = jnp.zeros((16,), jnp.float32)

    def zrow(i, carry):
        for k in range(8):
            rows_v[0, i, pl.ds(k * 16, 16)] = zero16
        return carry

    lax.fori_loop(0, CH, zrow, 0)

    def zb(i, carry):
        zbuf_v[pl.ds(i * 16, 16)] = zero16
        return carry

    lax.fori_loop(0, RPT // 16, zb, 0)

    def zacc(k, carry):
        pltpu.sync_copy(rows_v.at[0, pl.ds(0, CPR)],
                        acc_sh.at[pl.ds(rbase + k * CPR, CPR)])
        return carry

    lax.fori_loop(0, RPT // CPR, zacc, 0)
    pltpu.sync_copy(zbuf_v, den_sh.at[pl.ds(rbase, RPT)])

    pltpu.sync_copy(av_hbm.at[0], as_v)
    pltpu.sync_copy(av_hbm.at[1], ad_v)

    plsc.subcore_barrier()

    lanes = lax.broadcasted_iota(jnp.int32, (16,), 0)

    def sidx_copy(j, b):
        return pltpu.make_async_copy(
            src_hbm.at[pl.ds(ebase + j * CH, CH)], sidx[b], sem_is.at[b])

    def didx_copy(j, b):
        return pltpu.make_async_copy(
            dst_hbm.at[pl.ds(ebase + j * CH, CH)], didx[b], sem_id.at[b])

    def gather(b):
        return pltpu.make_async_copy(
            h_hbm.at[sidx[b]], rows_v.at[b], sem_g.at[b])

    def rscat(b):
        return pltpu.make_async_copy(
            rows_v.at[b], acc_sh.at[didx[b]], sem_s.at[b])

    def dscat(b):
        return pltpu.make_async_copy(
            e_v.at[pl.ds(b * CH, CH)], den_sh.at[didx[b]], sem_d.at[b])

    sidx_copy(0, 0).start()
    didx_copy(0, 0).start()
    sidx_copy(1, 1).start()
    didx_copy(1, 1).start()
    sidx_copy(0, 0).wait()
    gather(0).start()

    def chunk_arm(j, b):
        nb = 1 - b
        didx_copy(j, b).wait()
        for g in range(CH // 16):
            sl = pl.ds(g * 16, 16)
            a = (plsc.load_gather(as_v, [sidx[b][sl]])
                 + plsc.load_gather(ad_v, [didx[b][sl]]))
            a = jnp.maximum(a, 0.2 * a)
            e = jnp.exp(a)
            gid = ebase + j * CH + g * 16 + lanes
            e_v[pl.ds(b * CH + g * 16, 16)] = jnp.where(gid < E_TOT, e, 0.0)
        gather(b).wait()

        @pl.when(j + 2 < NCH)
        def _():
            sidx_copy(j + 2, b).start()

        @pl.when(j > 0)
        def _():
            rscat(nb).wait()
            dscat(nb).wait()

        @pl.when(jnp.logical_and(j > 0, j + 1 < NCH))
        def _():
            didx_copy(j + 1, nb).start()

        @pl.when(j + 1 < NCH)
        def _():
            sidx_copy(j + 1, nb).wait()
            gather(nb).start()

        def sgroup(g, carry):
            for t in range(16):
                es = plsc.load_gather(
                    e_v, [jnp.full((16,), b * CH + g * 16 + t, jnp.int32)])
                row = g * 16 + t
                for k in range(8):
                    sl = pl.ds(k * 16, 16)
                    rows_v[b, row, sl] = rows_v[b, row, sl] * es
            return carry

        lax.fori_loop(0, CH // 16, sgroup, 0)
        rscat(b).start(add=True)
        dscat(b).start(add=True)

    def chunk_body(j, carry):
        @pl.when(lax.rem(j, 2) == 0)
        def _():
            chunk_arm(j, 0)

        @pl.when(lax.rem(j, 2) == 1)
        def _():
            chunk_arm(j, 1)

        return carry

    lax.fori_loop(0, NCH, chunk_body, 0)
    rscat((NCH - 1) % 2).wait()
    dscat((NCH - 1) % 2).wait()

    plsc.subcore_barrier()

    def cpout(k, carry):
        pltpu.sync_copy(acc_sh.at[pl.ds(rbase + k * CPR, CPR)],
                        rows_v.at[0, pl.ds(0, CPR)])
        pltpu.sync_copy(rows_v.at[0, pl.ds(0, CPR)],
                        acc_out.at[c, pl.ds(rbase + k * CPR, CPR)])
        return carry

    lax.fori_loop(0, RPT // CPR, cpout, 0)
    pltpu.sync_copy(den_sh.at[pl.ds(rbase, RPT)], zbuf_v)
    pltpu.sync_copy(zbuf_v, den_out.at[c, pl.ds(rbase, RPT)])


def kernel(x, edge_index, W_gcn, a_s_gcn, a_d_gcn, b_gcn, W_r1, g1, be1,
           W_tr, a_s_tr, a_d_tr, b_tr, W_r2, g2, be2, W_p, b_p):
    loop = jnp.arange(N, dtype=jnp.int32)
    pad = (E_TOT + jnp.arange(E_PAD - E_TOT, dtype=jnp.int32)) % N
    src = jnp.concatenate([edge_index[0].astype(jnp.int32), loop, pad])
    dst = jnp.concatenate([edge_index[1].astype(jnp.int32), loop, pad])
    h1p, av1, res1 = _dense1(x, W_gcn, a_s_gcn, a_d_gcn, W_r1)
    acc1, den1 = _sc_aggregate(src, dst, av1, h1p)
    h1, h2p, av2, res2 = _mid(acc1, den1, b_gcn, res1, g1, be1,
                              W_tr, a_s_tr, a_d_tr, W_r2)
    acc2, den2 = _sc_aggregate(src, dst, av2, h2p)
    return _final(acc2, den2, b_tr, res2, g2, be2, h1, W_p, b_p)


# speedup vs baseline: 54.6364x; 1.0412x over previous
import functools

import jax
import jax.numpy as jnp
from jax import lax
from jax.experimental import pallas as pl
from jax.experimental.pallas import tpu as pltpu
from jax.experimental.pallas import tpu_sc as plsc

N = 10000
F = 128
E_RAW = 320000
E_TOT = E_RAW + N
NW = 32
CH = 128
NCH = 81
CPR = 64
EPT = NCH * CH
E_PAD = EPT * NW
ACC_N = 10240
RPT = ACC_N // 16


def _pack_logits(a_src, a_dst):
    lo = lax.bitcast_convert_type(a_src.astype(jnp.bfloat16),
                                  jnp.uint16).astype(jnp.uint32)
    hi = lax.bitcast_convert_type(a_dst.astype(jnp.bfloat16),
                                  jnp.uint16).astype(jnp.uint32)
    return lax.bitcast_convert_type(lo | (hi << 16), jnp.int32)


def _dense1_body(x_ref, wg_ref, as_ref, ad_ref, wr1_ref, h_ref, pk_ref, res_ref):
    x = x_ref[...]
    h = lax.dot_general(x, wg_ref[...], (((1,), (1,)), ((), ())),
                        preferred_element_type=jnp.float32)
    h_ref[...] = h
    pk_ref[...] = _pack_logits(jnp.sum(h * as_ref[...][None, :], axis=1),
                               jnp.sum(h * ad_ref[...][None, :], axis=1))
    res_ref[...] = lax.dot_general(x, wr1_ref[...], (((1,), (1,)), ((), ())),
                                   preferred_element_type=jnp.float32)


_dense1 = pl.pallas_call(
    _dense1_body,
    out_shape=[
        jax.ShapeDtypeStruct((N, F), jnp.float32),
        jax.ShapeDtypeStruct((N,), jnp.int32),
        jax.ShapeDtypeStruct((N, F), jnp.float32),
    ],
)


def _mid_body(acc_ref, den_ref, bg_ref, res1_ref, g1_ref, be1_ref, wtr_ref,
              astr_ref, adtr_ref, wr2_ref, h1_ref, h2p_ref, av2_ref, res2_ref):
    agg = acc_ref[0, :N, :] + acc_ref[1, :N, :]
    den = den_ref[0, :N] + den_ref[1, :N]
    t = agg / den[:, None] + bg_ref[...][None, :] + res1_ref[...]
    mu = jnp.mean(t, axis=0)
    var = jnp.mean((t - mu[None, :]) ** 2, axis=0)
    bn = (g1_ref[...][None, :] * (t - mu[None, :])
          / jnp.sqrt(var + 1e-5)[None, :] + be1_ref[...][None, :])
    h1 = bn + res1_ref[...]
    h1_ref[...] = h1
    h2p = lax.dot_general(h1, wtr_ref[...], (((1,), (1,)), ((), ())),
                          preferred_element_type=jnp.float32)
    h2p_ref[...] = h2p
    av2_ref[...] = _pack_logits(jnp.sum(h2p * astr_ref[...][None, :], axis=1),
                                jnp.sum(h2p * adtr_ref[...][None, :], axis=1))
    res2_ref[...] = lax.dot_general(h1, wr2_ref[...], (((1,), (1,)), ((), ())),
                                    preferred_element_type=jnp.float32)


_mid = pl.pallas_call(
    _mid_body,
    out_shape=[
        jax.ShapeDtypeStruct((N, F), jnp.float32),
        jax.ShapeDtypeStruct((N, F), jnp.float32),
        jax.ShapeDtypeStruct((N,), jnp.int32),
        jax.ShapeDtypeStruct((N, F), jnp.float32),
    ],
)


def _final_body(acc_ref, den_ref, btr_ref, res2_ref, g2_ref, be2_ref, h1_ref,
                wp_ref, bp_ref, out_ref):
    agg = acc_ref[0, :N, :] + acc_ref[1, :N, :]
    den = den_ref[0, :N] + den_ref[1, :N]
    t = agg / den[:, None] + btr_ref[...][None, :] + res2_ref[...]
    mu = jnp.mean(t, axis=0)
    var = jnp.mean((t - mu[None, :]) ** 2, axis=0)
    h2 = (g2_ref[...][None, :] * (t - mu[None, :])
          / jnp.sqrt(var + 1e-5)[None, :] + be2_ref[...][None, :]
          + res2_ref[...])
    hcat = jnp.concatenate([h1_ref[...], h2], axis=1)
    z = lax.dot_general(hcat, wp_ref[...], (((1,), (1,)), ((), ())),
                        preferred_element_type=jnp.float32) + bp_ref[...][None, :]
    out_ref[...] = jnp.where(z > 0, z, jnp.exp(z) - 1.0)


_final = pl.pallas_call(
    _final_body,
    out_shape=jax.ShapeDtypeStruct((N, F), jnp.float32),
)


@functools.partial(
    pl.kernel,
    mesh=plsc.VectorSubcoreMesh(core_axis_name="c", subcore_axis_name="s"),
    compiler_params=pltpu.CompilerParams(needs_layout_passes=False),
    out_type=[
        jax.ShapeDtypeStruct((2, ACC_N, F), jnp.float32),
        jax.ShapeDtypeStruct((2, ACC_N), jnp.float32),
    ],
    scratch_types=[
        pltpu.VMEM((N,), jnp.int32),
        pltpu.VMEM((2, 2 * CH), jnp.int32),
        pltpu.VMEM((CH,), jnp.int32),
        pltpu.VMEM((CH,), jnp.int32),
        pltpu.VMEM((2 * CH,), jnp.float32),
        pltpu.VMEM((2, CH, F), jnp.float32),
        pltpu.VMEM((RPT,), jnp.float32),
        pltpu.VMEM_SHARED((ACC_N, F), jnp.float32),
        pltpu.VMEM_SHARED((ACC_N,), jnp.float32),
        pltpu.SemaphoreType.DMA((2,)),
        pltpu.SemaphoreType.DMA((2,)),
        pltpu.SemaphoreType.DMA((2,)),
        pltpu.SemaphoreType.DMA((2,)),
    ],
)
def _sc_aggregate(eidx_hbm, pk_hbm, h_hbm, acc_out, den_out,
                  pk_v, idx_v, didx0, didx1, e_v, rows_v, zbuf_v,
                  acc_sh, den_sh, sem_i, sem_g, sem_s, sem_d):
    c = lax.axis_index("c")
    s = lax.axis_index("s")
    wid = c * 16 + s
    ebase = wid * EPT
    rbase = s * RPT
    didx = (didx0, didx1)

    zero16 = jnp.zeros((16,), jnp.float32)

    def zrow(i, carry):
        for k in range(8):
            rows_v[0, i, pl.ds(k * 16, 16)] = zero16
        return carry

    lax.fori_loop(0, CPR, zrow, 0)

    def zb(i, carry):
        zbuf_v[pl.ds(i * 16, 16)] = zero16
        return carry

    lax.fori_loop(0, RPT // 16, zb, 0)

    def zacc(k, carry):
        pltpu.sync_copy(rows_v.at[0, pl.ds(0, CPR)],
                        acc_sh.at[pl.ds(rbase + k * CPR, CPR)])
        return carry

    lax.fori_loop(0, RPT // CPR, zacc, 0)
    pltpu.sync_copy(zbuf_v, den_sh.at[pl.ds(rbase, RPT)])

    pltpu.sync_copy(pk_hbm, pk_v)

    plsc.subcore_barrier()

    lanes = lax.broadcasted_iota(jnp.int32, (16,), 0)

    def idx_copy(j, b):
        return pltpu.make_async_copy(
            eidx_hbm.at[pl.ds((wid * NCH + j) * 2 * CH, 2 * CH)],
            idx_v.at[b], sem_i.at[b])

    def gather(b):
        return pltpu.make_async_copy(
            h_hbm.at[idx_v.at[b, pl.ds(0, CH)]], rows_v.at[b], sem_g.at[b])

    def rscat(b):
        return pltpu.make_async_copy(
            rows_v.at[b], acc_sh.at[didx[b]], sem_s.at[b])

    def dscat(b):
        return pltpu.make_async_copy(
            e_v.at[pl.ds(b * CH, CH)], den_sh.at[didx[b]], sem_d.at[b])

    idx_copy(0, 0).start()
    idx_copy(1, 1).start()
    idx_copy(0, 0).wait()
    gather(0).start()

    def chunk_arm(j, b):
        nb = 1 - b
        for g in range(CH // 16):
            sl = pl.ds(g * 16, 16)
            si = idx_v[b, pl.ds(g * 16, 16)]
            di = idx_v[b, pl.ds(CH + g * 16, 16)]
            didx[b][sl] = di
            gs = plsc.load_gather(pk_v, [si])
            gd = plsc.load_gather(pk_v, [di])
            a = (plsc.bitcast(lax.shift_left(gs, 16), jnp.float32)
                 + plsc.bitcast(gd & jnp.int32(-65536), jnp.float32))
            a = jnp.maximum(a, 0.2 * a)
            e = jnp.exp(a)
            gid = ebase + j * CH + g * 16 + lanes
            e_v[pl.ds(b * CH + g * 16, 16)] = jnp.where(gid < E_TOT, e, 0.0)
        gather(b).wait()

        @pl.when(j + 2 < NCH)
        def _():
            idx_copy(j + 2, b).start()

        @pl.when(j > 0)
        def _():
            rscat(nb).wait()
            dscat(nb).wait()

        @pl.when(j + 1 < NCH)
        def _():
            idx_copy(j + 1, nb).wait()
            gather(nb).start()

        def sgroup(g, carry):
            for t in range(16):
                es = plsc.load_gather(
                    e_v, [jnp.full((16,), b * CH + g * 16 + t, jnp.int32)])
                row = g * 16 + t
                for k in range(8):
                    sl = pl.ds(k * 16, 16)
                    rows_v[b, row, sl] = rows_v[b, row, sl] * es
            return carry

        lax.fori_loop(0, CH // 16, sgroup, 0)
        rscat(b).start(add=True)
        dscat(b).start(add=True)

    def chunk_body(j, carry):
        @pl.when(lax.rem(j, 2) == 0)
        def _():
            chunk_arm(j, 0)

        @pl.when(lax.rem(j, 2) == 1)
        def _():
            chunk_arm(j, 1)

        return carry

    lax.fori_loop(0, NCH, chunk_body, 0)
    rscat((NCH - 1) % 2).wait()
    dscat((NCH - 1) % 2).wait()

    plsc.subcore_barrier()

    def cpout(k, carry):
        pltpu.sync_copy(acc_sh.at[pl.ds(rbase + k * CPR, CPR)],
                        rows_v.at[0, pl.ds(0, CPR)])
        pltpu.sync_copy(rows_v.at[0, pl.ds(0, CPR)],
                        acc_out.at[c, pl.ds(rbase + k * CPR, CPR)])
        return carry

    lax.fori_loop(0, RPT // CPR, cpout, 0)
    pltpu.sync_copy(den_sh.at[pl.ds(rbase, RPT)], zbuf_v)
    pltpu.sync_copy(zbuf_v, den_out.at[c, pl.ds(rbase, RPT)])


def kernel(x, edge_index, W_gcn, a_s_gcn, a_d_gcn, b_gcn, W_r1, g1, be1,
           W_tr, a_s_tr, a_d_tr, b_tr, W_r2, g2, be2, W_p, b_p):
    loop = jnp.arange(N, dtype=jnp.int32)
    pad = (E_TOT + jnp.arange(E_PAD - E_TOT, dtype=jnp.int32)) % N
    src = jnp.concatenate([edge_index[0].astype(jnp.int32), loop, pad])
    dst = jnp.concatenate([edge_index[1].astype(jnp.int32), loop, pad])
    eidx = jnp.stack([src.reshape(NW * NCH, CH),
                      dst.reshape(NW * NCH, CH)], axis=1).reshape(-1)

    h1p, pk1, res1 = _dense1(x, W_gcn, a_s_gcn, a_d_gcn, W_r1)
    acc1, den1 = _sc_aggregate(eidx, pk1, h1p)
    h1, h2p, pk2, res2 = _mid(acc1, den1, b_gcn, res1, g1, be1,
                              W_tr, a_s_tr, a_d_tr, W_r2)
    acc2, den2 = _sc_aggregate(eidx, pk2, h2p)
    return _final(acc2, den2, b_tr, res2, g2, be2, h1, W_p, b_p)
